# Initial kernel scaffold; baseline (speedup 1.0000x reference)
import jax
import jax.numpy as jnp
from jax.experimental import pallas as pl


def kernel(concepts, ages, abspos, segments, concept_table, age_table, abspos_table, segment_table, ln_gamma, ln_beta):
    raise NotImplementedError("write your pallas kernel here")


# speedup vs baseline: 8.0059x; 8.0059x over previous
import functools

import jax
import jax.numpy as jnp
from jax import lax
from jax.experimental import pallas as pl
from jax.experimental.pallas import tpu as pltpu
from jax.experimental.pallas import tpu_sc as plsc

H = 64
NC, NS = 2, 16
NW = NC * NS
CHUNK = 128
LN_EPS = 1e-12


def _combine_body(a_ref, s_ref, o_ref):
    o_ref[...] = a_ref[...] + s_ref[...]


def _combine_tables(abspos_table, segment_table):
    n = abspos_table.shape[0]
    blk = 1000
    return pl.pallas_call(
        _combine_body,
        out_shape=jax.ShapeDtypeStruct(abspos_table.shape, abspos_table.dtype),
        grid=(n // blk,),
        in_specs=[
            pl.BlockSpec((blk, H), lambda i: (i, 0)),
            pl.BlockSpec((blk, H), lambda i: (i, 0)),
        ],
        out_specs=pl.BlockSpec((blk, H), lambda i: (i, 0)),
    )(abspos_table, segment_table)


def _rsqrt16(x):
    i = plsc.bitcast(x, jnp.int32)
    y = plsc.bitcast(jnp.int32(0x5F3759DF) - (i >> 1), jnp.float32)
    for _ in range(3):
        y = y * (1.5 - 0.5 * x * y * y)
    return y


def _sc_kernel_body(T, cidx_hbm, pidx_hbm, aidx_hbm, ctab_hbm, gtab_hbm,
                    atab_hbm, gam_hbm, bet_hbm, out_hbm,
                    cidx_v, pidx_v, aidx_v, crow, grow, arow, orow,
                    gam_v, bet_v, isem, sem, osem):
    wid = lax.axis_index("s") * NC + lax.axis_index("c")
    base = wid * T

    pltpu.sync_copy(gam_hbm, gam_v)
    pltpu.sync_copy(bet_hbm, bet_v)

    gams = [gam_v[pl.ds(16 * k, 16)] for k in range(4)]
    bets = [bet_v[pl.ds(16 * k, 16)] for k in range(4)]

    num_chunks = T // CHUNK

    def chunk_body(ci, _):
        t0 = ci * CHUNK
        i1 = pltpu.async_copy(cidx_hbm.at[pl.ds(base + t0, CHUNK)], cidx_v, isem)
        i2 = pltpu.async_copy(pidx_hbm.at[pl.ds(base + t0, CHUNK)], pidx_v, isem)
        i3 = pltpu.async_copy(aidx_hbm.at[pl.ds(base + t0, CHUNK)], aidx_v, isem)
        i1.wait()
        i2.wait()
        i3.wait()
        c1 = pltpu.async_copy(ctab_hbm.at[cidx_v], crow, sem)
        c2 = pltpu.async_copy(gtab_hbm.at[pidx_v], grow, sem)
        c3 = pltpu.async_copy(atab_hbm.at[aidx_v], arow, sem)
        c1.wait()
        c2.wait()
        c3.wait()

        def tok(t, _):
            e = [crow[t, pl.ds(16 * k, 16)] + grow[t, pl.ds(16 * k, 16)]
                 + arow[t, pl.ds(16 * k, 16)] for k in range(4)]
            tot = jnp.sum((e[0] + e[1]) + (e[2] + e[3]))
            mean = tot * (1.0 / H)
            totsq = jnp.sum((e[0] * e[0] + e[1] * e[1])
                            + (e[2] * e[2] + e[3] * e[3]))
            var = jnp.maximum(totsq * (1.0 / H) - mean * mean, 0.0)
            rstd = _rsqrt16(jnp.full((16,), var + LN_EPS, jnp.float32))
            for k in range(4):
                orow[t, pl.ds(16 * k, 16)] = (e[k] - mean) * rstd * gams[k] + bets[k]
            return 0

        lax.fori_loop(0, CHUNK, tok, 0)
        pltpu.async_copy(orow, out_hbm.at[pl.ds(base + t0, CHUNK)], osem).wait()
        return 0

    lax.fori_loop(0, num_chunks, chunk_body, 0)


def kernel(concepts, ages, abspos, segments, concept_table, age_table,
           abspos_table, segment_table, ln_gamma, ln_beta):
    B, L = concepts.shape
    N = B * L
    T = N // NW

    gtab = _combine_tables(abspos_table, segment_table)

    cidx = concepts.reshape(N).astype(jnp.int32)
    pidx = abspos.reshape(N).astype(jnp.int32)
    aidx = ages.reshape(N).astype(jnp.int32)

    mesh = plsc.VectorSubcoreMesh(core_axis_name="c", subcore_axis_name="s")
    run = functools.partial(
        pl.kernel,
        out_type=jax.ShapeDtypeStruct((N, H), jnp.float32),
        mesh=mesh,
        compiler_params=pltpu.CompilerParams(needs_layout_passes=False, use_tc_tiling_on_sc=False),
        scratch_types=[
            pltpu.VMEM((CHUNK,), jnp.int32),
            pltpu.VMEM((CHUNK,), jnp.int32),
            pltpu.VMEM((CHUNK,), jnp.int32),
            pltpu.VMEM((CHUNK, H), jnp.float32),
            pltpu.VMEM((CHUNK, H), jnp.float32),
            pltpu.VMEM((CHUNK, H), jnp.float32),
            pltpu.VMEM((CHUNK, H), jnp.float32),
            pltpu.VMEM((H,), jnp.float32),
            pltpu.VMEM((H,), jnp.float32),
            pltpu.SemaphoreType.DMA,
            pltpu.SemaphoreType.DMA,
            pltpu.SemaphoreType.DMA,
        ],
    )(functools.partial(_sc_kernel_body, T))

    out = run(cidx, pidx, aidx, concept_table, gtab, age_table,
              ln_gamma, ln_beta)
    return out.reshape(B, L, H)
